# Initial kernel scaffold; baseline (speedup 1.0000x reference)
import jax
import jax.numpy as jnp
from jax.experimental import pallas as pl


def kernel(x, mask):
    raise NotImplementedError("write your pallas kernel here")


# speedup vs baseline: 4.7175x; 4.7175x over previous
import functools

import jax
import jax.numpy as jnp
from jax.experimental import pallas as pl
from jax.experimental.pallas import tpu as pltpu

_N = 320
_DROP_RATE = 0.8


def _threshold_kernel(mask_ref, bm_ref, *, k):
    m = mask_ref[...]
    b = jax.lax.bitcast_convert_type(m, jnp.int32)
    keys = b ^ ((b >> 31) & jnp.int32(0x7FFFFFFF))

    def count_ge(c):
        return jnp.sum((keys >= c).astype(jnp.int32))

    t0 = jnp.where(count_ge(jnp.int32(0)) >= k, jnp.int32(0),
                   jnp.int32(-2147483648))

    def body(i, t):
        cand = t + (jnp.int32(1) << (jnp.int32(30) - i))
        return jnp.where(count_ge(cand) >= k, cand, t)

    t = jax.lax.fori_loop(0, 31, body, t0)
    tb = t ^ ((t >> 31) & jnp.int32(0x7FFFFFFF))
    thr = jax.lax.bitcast_convert_type(tb, jnp.float32)
    bm_ref[...] = (m >= thr).astype(jnp.float32)


def _transform_kernel(x_ref, bm_ref, wcat_ref, wr_ref, wi_ref, ws_ref,
                      out_ref):
    n = out_ref.shape[-1]
    xm = x_ref[0] * bm_ref[...]
    tcat = jnp.dot(xm, wcat_ref[...], preferred_element_type=jnp.float32)
    tr = tcat[:, :n]
    ti = tcat[:, n:]
    q1 = jnp.dot(wr_ref[...], tr, preferred_element_type=jnp.float32)
    q2 = jnp.dot(wi_ref[...], ti, preferred_element_type=jnp.float32)
    q3 = jnp.dot(ws_ref[...], tr + ti, preferred_element_type=jnp.float32)
    yr = q1 - q2
    yi = q3 - q1 - q2
    out_ref[0] = jnp.sqrt(yr * yr + yi * yi)


def _idft_mats(n):
    idx = jnp.arange(n, dtype=jnp.int32)
    ang = (2.0 * jnp.pi / n) * ((idx[:, None] * idx[None, :]) % n).astype(
        jnp.float32)
    scale = 1.0 / jnp.sqrt(jnp.float32(n))
    wr = jnp.cos(ang) * scale
    wi = jnp.sin(ang) * scale
    ws = wr + wi
    wtop = jnp.stack([wr, -wi], axis=1).reshape(2 * n, n)
    wbot = jnp.stack([wi, wr], axis=1).reshape(2 * n, n)
    wcat = jnp.concatenate([wtop, wbot], axis=1)
    return wcat, wr, wi, ws


def kernel(x, mask):
    b = x.shape[0]
    n = x.shape[1]
    mask2d = mask.reshape(n, 2 * n)
    x3 = x.reshape(b, n, 2 * n)
    k = int((1.0 - _DROP_RATE) * mask.size)

    bm = pl.pallas_call(
        functools.partial(_threshold_kernel, k=k),
        out_shape=jax.ShapeDtypeStruct((n, 2 * n), jnp.float32),
    )(mask2d)

    wcat, wr, wi, ws = _idft_mats(n)

    out = pl.pallas_call(
        _transform_kernel,
        grid=(b,),
        in_specs=[
            pl.BlockSpec((1, n, 2 * n), lambda i: (i, 0, 0)),
            pl.BlockSpec((n, 2 * n), lambda i: (0, 0)),
            pl.BlockSpec((2 * n, 2 * n), lambda i: (0, 0)),
            pl.BlockSpec((n, n), lambda i: (0, 0)),
            pl.BlockSpec((n, n), lambda i: (0, 0)),
            pl.BlockSpec((n, n), lambda i: (0, 0)),
        ],
        out_specs=pl.BlockSpec((1, n, n), lambda i: (i, 0, 0)),
        out_shape=jax.ShapeDtypeStruct((b, n, n), jnp.float32),
        compiler_params=pltpu.CompilerParams(
            dimension_semantics=("arbitrary",),
        ),
    )(x3, bm, wcat, wr, wi, ws)

    return out[:, None, :, :]


# speedup vs baseline: 7.9221x; 1.6793x over previous
import functools

import jax
import jax.numpy as jnp
from jax.experimental import pallas as pl
from jax.experimental.pallas import tpu as pltpu

_N = 320
_DROP_RATE = 0.8


def _threshold_kernel(mask_ref, bm_ref, *, k):
    m = mask_ref[...]
    b = jax.lax.bitcast_convert_type(m, jnp.int32)
    keys = b ^ ((b >> 31) & jnp.int32(0x7FFFFFFF))

    def count_ge(c):
        return jnp.sum((keys >= c).astype(jnp.int32))

    t0 = jnp.where(count_ge(jnp.int32(0)) >= k, jnp.int32(0),
                   jnp.int32(-2147483648))

    def body(i, t):
        cand = t + (jnp.int32(1) << (jnp.int32(30) - i))
        return jnp.where(count_ge(cand) >= k, cand, t)

    t = jax.lax.fori_loop(0, 31, body, t0)
    tb = t ^ ((t >> 31) & jnp.int32(0x7FFFFFFF))
    thr = jax.lax.bitcast_convert_type(tb, jnp.float32)
    bm_ref[...] = (m >= thr).astype(jnp.float32)


def _transform_kernel(xr_ref, xi_ref, bmr_ref, bmi_ref, wr_ref, wi_ref,
                      ws_ref, out_ref):
    xr = xr_ref[0] * bmr_ref[...]
    xi = xi_ref[0] * bmi_ref[...]
    p1 = jnp.dot(xr, wr_ref[...], preferred_element_type=jnp.float32)
    p2 = jnp.dot(xi, wi_ref[...], preferred_element_type=jnp.float32)
    p3 = jnp.dot(xr + xi, ws_ref[...], preferred_element_type=jnp.float32)
    tr = p1 - p2
    ti = p3 - p1 - p2
    q1 = jnp.dot(wr_ref[...], tr, preferred_element_type=jnp.float32)
    q2 = jnp.dot(wi_ref[...], ti, preferred_element_type=jnp.float32)
    q3 = jnp.dot(ws_ref[...], tr + ti, preferred_element_type=jnp.float32)
    yr = q1 - q2
    yi = q3 - q1 - q2
    out_ref[0] = jnp.sqrt(yr * yr + yi * yi)


def _idft_mats(n):
    idx = jnp.arange(n, dtype=jnp.int32)
    ang = (2.0 * jnp.pi / n) * ((idx[:, None] * idx[None, :]) % n).astype(
        jnp.float32)
    scale = 1.0 / jnp.sqrt(jnp.float32(n))
    wr = jnp.cos(ang) * scale
    wi = jnp.sin(ang) * scale
    ws = wr + wi
    return wr, wi, ws


def kernel(x, mask):
    b = x.shape[0]
    n = x.shape[1]
    mask2d = mask.reshape(n, 2 * n)
    k = int((1.0 - _DROP_RATE) * mask.size)

    bm = pl.pallas_call(
        functools.partial(_threshold_kernel, k=k),
        out_shape=jax.ShapeDtypeStruct((n, 2 * n), jnp.float32),
    )(mask2d)
    bm4 = bm.reshape(1, n, n, 2)
    bmr = bm4[0, :, :, 0]
    bmi = bm4[0, :, :, 1]
    xr = x[:, :, :, 0]
    xi = x[:, :, :, 1]

    wr, wi, ws = _idft_mats(n)

    out = pl.pallas_call(
        _transform_kernel,
        grid=(b,),
        in_specs=[
            pl.BlockSpec((1, n, n), lambda i: (i, 0, 0)),
            pl.BlockSpec((1, n, n), lambda i: (i, 0, 0)),
            pl.BlockSpec((n, n), lambda i: (0, 0)),
            pl.BlockSpec((n, n), lambda i: (0, 0)),
            pl.BlockSpec((n, n), lambda i: (0, 0)),
            pl.BlockSpec((n, n), lambda i: (0, 0)),
            pl.BlockSpec((n, n), lambda i: (0, 0)),
        ],
        out_specs=pl.BlockSpec((1, n, n), lambda i: (i, 0, 0)),
        out_shape=jax.ShapeDtypeStruct((b, n, n), jnp.float32),
        compiler_params=pltpu.CompilerParams(
            dimension_semantics=("arbitrary",),
        ),
    )(xr, xi, bmr, bmi, wr, wi, ws)

    return out[:, None, :, :]
